# baseline (speedup 1.0000x reference)
import jax
import jax.numpy as jnp
from jax.experimental import pallas as pl
from jax.experimental.pallas import tpu as pltpu

B, DK, SEQ_DIM, HID_DIM, N, NE_DIM = 32, 512, 4096, 512, 2048, 32
NB = 8
BN = N // NB
NSTEP = 2 * NB
INV_SQRT_S = 1.0 / (float(SEQ_DIM) ** 0.5)


def _bitplane(xi, j, out_dtype):
    return ((xi >> (7 - j)) & 1).astype(out_dtype)


def _unpack_bitmajor(xi):
    return jnp.concatenate(
        [_bitplane(xi, j, jnp.float32) for j in range(8)], axis=1)


def _plane_copy(step, wres_hbm, wk_hbm, wbuf, wsem):
    slot = jax.lax.rem(step, 2)

    @pl.when(step < NB)
    def _():
        pltpu.make_async_copy(
            wres_hbm.at[:, step, :], wbuf.at[slot], wsem.at[slot]).start()

    @pl.when(jnp.logical_and(step >= NB, step < NSTEP))
    def _():
        pltpu.make_async_copy(
            wk_hbm.at[:, step - NB, :], wbuf.at[slot], wsem.at[slot]).start()


def _seqnet_kernel(q_ref, qok_ref, refs_ref, mask_ref, nst_ref,
                   wres_hbm, wk_hbm, ne_ref, bo_ref, out_ref,
                   a1_s, qproj_s, ku_s, scores_s, g_s, wbuf, wsem):
    i = pl.program_id(0)

    @pl.when(i == 0)
    def _first_fetch():
        _plane_copy(0, wres_hbm, wk_hbm, wbuf, wsem)
    _plane_copy(i + 1, wres_hbm, wk_hbm, wbuf, wsem)

    slot = jax.lax.rem(i, 2)
    pltpu.make_async_copy(wbuf.at[slot], wbuf.at[slot], wsem.at[slot]).wait()
    wplane = wbuf[slot]

    @pl.when(i < NB)
    def _phase1():
        qi = q_ref[...].astype(jnp.int32)

        @pl.when(i == 0)
        def _init_a1():
            qoki = qok_ref[...].astype(jnp.int32)
            for j in range(8):
                a1_s[:, j * DK:(j + 1) * DK] = (
                    _bitplane(qi, j, jnp.float32)
                    * _bitplane(qoki, j, jnp.float32) * INV_SQRT_S)

        qplane_i = ((qi >> (7 - i)) & 1).astype(jnp.float32)
        contrib = jnp.dot(qplane_i, wplane,
                          preferred_element_type=jnp.float32)

        @pl.when(i == 0)
        def _qp0():
            qproj_s[...] = contrib

        @pl.when(i > 0)
        def _qpn():
            qproj_s[...] = qproj_s[...] + contrib

        kb = _unpack_bitmajor(
            refs_ref[pl.ds(i * BN, BN), :].astype(jnp.int32))
        ku_s[pl.ds(i * BN, BN), :] = kb
        scores_s[:, pl.ds(i * BN, BN)] = jax.lax.dot_general(
            a1_s[...], kb, (((1,), (1,)), ((), ())),
            preferred_element_type=jnp.float32)

    @pl.when(i >= NB)
    def _phase2():
        vj = jax.lax.dot_general(qproj_s[...], wplane,
                                 (((1,), (1,)), ((), ())),
                                 preferred_element_type=jnp.float32)
        gj = jax.lax.dot_general(vj, ku_s[:, pl.ds((i - NB) * DK, DK)],
                                 (((1,), (1,)), ((), ())),
                                 preferred_element_type=jnp.float32)

        @pl.when(i == NB)
        def _g0():
            g_s[...] = gj

        @pl.when(i > NB)
        def _gn():
            g_s[...] = g_s[...] + gj

    @pl.when(i == NSTEP - 1)
    def _epilogue():
        s = jnp.where(mask_ref[...] > 0, scores_s[...], -1e9)
        m = jnp.max(s, axis=1, keepdims=True)
        e = jnp.exp(s - m)
        w = e / jnp.sum(e, axis=1, keepdims=True)
        ne0 = jnp.sum(ne_ref[0:1, :], keepdims=True)
        ne1 = jnp.sum(ne_ref[1:2, :], keepdims=True)
        c = ne0 * nst_ref[0:1, :] + ne1 * nst_ref[1:2, :]
        out_ref[...] = w * g_s[...] + c + bo_ref[...]


@jax.jit
def kernel(Q, Q_ok, td_refs, td_mask, td_node_state, W_res, W_k, node_embed, b_o):
    wres_3d = W_res.reshape(DK, 8, HID_DIM)
    wk_3d = W_k.reshape(DK, 8, HID_DIM)
    nst_t = td_node_state.T
    mask_f = td_mask.astype(jnp.float32)

    full = lambda shape: pl.BlockSpec(shape, lambda i: (0,) * len(shape))
    out = pl.pallas_call(
        _seqnet_kernel,
        grid=(NSTEP,),
        in_specs=[
            full((B, DK)),
            full((B, DK)),
            full((N, DK)),
            full((B, N)),
            full((2, N)),
            pl.BlockSpec(memory_space=pltpu.MemorySpace.HBM),
            pl.BlockSpec(memory_space=pltpu.MemorySpace.HBM),
            full((2, NE_DIM)),
            full((1, N)),
        ],
        out_specs=full((B, N)),
        out_shape=jax.ShapeDtypeStruct((B, N), jnp.float32),
        scratch_shapes=[
            pltpu.VMEM((B, SEQ_DIM), jnp.float32),
            pltpu.VMEM((B, HID_DIM), jnp.float32),
            pltpu.VMEM((N, SEQ_DIM), jnp.float32),
            pltpu.VMEM((B, N), jnp.float32),
            pltpu.VMEM((B, N), jnp.float32),
            pltpu.VMEM((2, DK, HID_DIM), jnp.float32),
            pltpu.SemaphoreType.DMA((2,)),
        ],
    )(Q, Q_ok, td_refs, mask_f, nst_t, wres_3d, wk_3d, node_embed, b_o)
    return out


# speedup vs baseline: 1.1706x; 1.1706x over previous
import jax
import jax.numpy as jnp
from jax.experimental import pallas as pl
from jax.experimental.pallas import tpu as pltpu

B, DK, SEQ_DIM, HID_DIM, N, NE_DIM = 32, 512, 4096, 512, 2048, 32
NSTEP = 8
RB = N // 4
INV_SQRT_S = 1.0 / (float(SEQ_DIM) ** 0.5)


def _bitplane(xi, j, out_dtype):
    return ((xi >> (7 - j)) & 1).astype(out_dtype)


def _unpack_bitmajor(xi):
    return jnp.concatenate(
        [_bitplane(xi, j, jnp.float32) for j in range(8)], axis=1)


def _seqnet_kernel(q_ref, qok_ref, refs_ref, mask_ref, nst_ref,
                   wres_hbm, wk_hbm, ne_ref, bo_ref, out_ref,
                   a1_s, qproj_s, ku_s, scores_s, g_s, wbuf, wsem):
    i = pl.program_id(0)

    @pl.when(i == 0)
    def _fetch_all_planes():
        for s in range(8):
            pltpu.make_async_copy(
                wres_hbm.at[:, s, :], wbuf.at[s], wsem.at[s]).start()
        for s in range(8):
            pltpu.make_async_copy(
                wk_hbm.at[:, s, :], wbuf.at[8 + s], wsem.at[8 + s]).start()

    s0, s1 = 2 * i, 2 * i + 1
    pltpu.make_async_copy(wbuf.at[s0], wbuf.at[s0], wsem.at[s0]).wait()
    pltpu.make_async_copy(wbuf.at[s1], wbuf.at[s1], wsem.at[s1]).wait()
    w0, w1 = wbuf[s0], wbuf[s1]

    @pl.when(i < 4)
    def _phase1():
        qi = q_ref[...].astype(jnp.int32)

        @pl.when(i == 0)
        def _init_a1():
            qoki = qok_ref[...].astype(jnp.int32)
            for j in range(8):
                a1_s[:, j * DK:(j + 1) * DK] = (
                    _bitplane(qi, j, jnp.float32)
                    * _bitplane(qoki, j, jnp.float32) * INV_SQRT_S)

        qp0 = ((qi >> (7 - s0)) & 1).astype(jnp.float32)
        qp1 = ((qi >> (7 - s1)) & 1).astype(jnp.float32)
        contrib = (jnp.dot(qp0, w0, preferred_element_type=jnp.float32)
                   + jnp.dot(qp1, w1, preferred_element_type=jnp.float32))

        @pl.when(i == 0)
        def _qp0():
            qproj_s[...] = contrib

        @pl.when(i > 0)
        def _qpn():
            qproj_s[...] = qproj_s[...] + contrib

        kb = _unpack_bitmajor(
            refs_ref[pl.ds(i * RB, RB), :].astype(jnp.int32))
        ku_s[pl.ds(i * RB, RB), :] = kb
        scores_s[:, pl.ds(i * RB, RB)] = jax.lax.dot_general(
            a1_s[...], kb, (((1,), (1,)), ((), ())),
            preferred_element_type=jnp.float32)

    @pl.when(i >= 4)
    def _phase2():
        v0 = jax.lax.dot_general(qproj_s[...], w0, (((1,), (1,)), ((), ())),
                                 preferred_element_type=jnp.float32)
        v1 = jax.lax.dot_general(qproj_s[...], w1, (((1,), (1,)), ((), ())),
                                 preferred_element_type=jnp.float32)
        vcat = jnp.concatenate([v0, v1], axis=1)
        gj = jax.lax.dot_general(
            vcat, ku_s[:, pl.ds((i - 4) * (2 * DK), 2 * DK)],
            (((1,), (1,)), ((), ())),
            preferred_element_type=jnp.float32)

        @pl.when(i == 4)
        def _g0():
            g_s[...] = gj

        @pl.when(i > 4)
        def _gn():
            g_s[...] = g_s[...] + gj

    @pl.when(i == NSTEP - 1)
    def _epilogue():
        s = jnp.where(mask_ref[...] > 0, scores_s[...], -1e9)
        m = jnp.max(s, axis=1, keepdims=True)
        e = jnp.exp(s - m)
        w = e / jnp.sum(e, axis=1, keepdims=True)
        ne0 = jnp.sum(ne_ref[0:1, :], keepdims=True)
        ne1 = jnp.sum(ne_ref[1:2, :], keepdims=True)
        c = ne0 * nst_ref[0:1, :] + ne1 * nst_ref[1:2, :]
        out_ref[...] = w * g_s[...] + c + bo_ref[...]


@jax.jit
def kernel(Q, Q_ok, td_refs, td_mask, td_node_state, W_res, W_k, node_embed, b_o):
    wres_3d = W_res.reshape(DK, 8, HID_DIM)
    wk_3d = W_k.reshape(DK, 8, HID_DIM)
    nst_t = td_node_state.T
    mask_f = td_mask.astype(jnp.float32)

    full = lambda shape: pl.BlockSpec(shape, lambda i: (0,) * len(shape))
    out = pl.pallas_call(
        _seqnet_kernel,
        grid=(NSTEP,),
        in_specs=[
            full((B, DK)),
            full((B, DK)),
            full((N, DK)),
            full((B, N)),
            full((2, N)),
            pl.BlockSpec(memory_space=pltpu.MemorySpace.HBM),
            pl.BlockSpec(memory_space=pltpu.MemorySpace.HBM),
            full((2, NE_DIM)),
            full((1, N)),
        ],
        out_specs=full((B, N)),
        out_shape=jax.ShapeDtypeStruct((B, N), jnp.float32),
        scratch_shapes=[
            pltpu.VMEM((B, SEQ_DIM), jnp.float32),
            pltpu.VMEM((B, HID_DIM), jnp.float32),
            pltpu.VMEM((N, SEQ_DIM), jnp.float32),
            pltpu.VMEM((B, N), jnp.float32),
            pltpu.VMEM((B, N), jnp.float32),
            pltpu.VMEM((16, DK, HID_DIM), jnp.float32),
            pltpu.SemaphoreType.DMA((16,)),
        ],
    )(Q, Q_ok, td_refs, mask_f, nst_t, wres_3d, wk_3d, node_embed, b_o)
    return out
